# baseline (speedup 1.0000x reference)
import functools
import math

import jax
import jax.numpy as jnp
from jax import lax
from jax.experimental import pallas as pl
from jax.experimental.pallas import tpu as pltpu
from jax.experimental.pallas import tpu_sc as plsc

NC = 2
NS = 16
NW = NC * NS
L = 16

D_MODEL = 1024
SCALE = math.sqrt(D_MODEL)

CHUNK = 16
NBUF = 4


@functools.partial(jax.jit, static_argnums=(2, 3, 4))
def _gather_scale(table, x, R, C, D):
    B = R * C
    b_per_w = B // NW
    w_per_r = C // b_per_w
    NCHUNK = b_per_w // CHUNK
    NGRP = NCHUNK // NBUF
    mesh = plsc.VectorSubcoreMesh(core_axis_name="c", subcore_axis_name="s")

    @functools.partial(
        pl.kernel,
        out_type=jax.ShapeDtypeStruct((B, D), jnp.float32),
        mesh=mesh,
        scratch_types=(
            [pltpu.VMEM((b_per_w,), jnp.int32)]
            + [pltpu.VMEM((CHUNK, D), jnp.float32)] * NBUF
            + [pltpu.SemaphoreType.DMA] * (2 * NBUF + 2)
        ),
    )
    def body(table_hbm, idx_hbm, out_hbm, idx_v, *rest):
        bufs = rest[:NBUF]
        gsems = rest[NBUF:2 * NBUF]
        osems = rest[2 * NBUF:3 * NBUF]
        isem0, isem1 = rest[3 * NBUF:]
        wid = lax.axis_index("s") * NC + lax.axis_index("c")
        base = wid * b_per_w
        xr = wid // w_per_r
        xc = (wid % w_per_r) * b_per_w
        HALF = b_per_w // 2
        di0 = pltpu.make_async_copy(
            idx_hbm.at[xr, pl.ds(xc, HALF)], idx_v.at[pl.ds(0, HALF)], isem0)
        di1 = pltpu.make_async_copy(
            idx_hbm.at[xr, pl.ds(xc + HALF, HALF)],
            idx_v.at[pl.ds(HALF, HALF)], isem1)
        di0.start()
        di1.start()

        def gather_desc(c, k):
            return pltpu.make_async_copy(
                table_hbm.at[idx_v.at[pl.ds(c * CHUNK, CHUNK)]],
                bufs[k], gsems[k])

        def out_desc(c, k):
            return pltpu.make_async_copy(
                bufs[k], out_hbm.at[pl.ds(base + c * CHUNK, CHUNK)], osems[k])

        H = CHUNK // 2

        def out_half_desc(c, k, h):
            return pltpu.make_async_copy(
                bufs[k].at[pl.ds(h * H, H)],
                out_hbm.at[pl.ds(base + c * CHUNK + h * H, H)], osems[k])

        def scale_half(buf, h):
            @plsc.parallel_loop(h * H, (h + 1) * H)
            def _(r):
                for j in range(D // L):
                    sl = (r, pl.ds(j * L, L))
                    buf[sl] = buf[sl] * SCALE

        di0.wait()
        gather_desc(0, 0).start()
        gather_desc(1, 1).start()
        di1.wait()

        def group(g, _):
            for k in range(NBUF):
                c = g * NBUF + k
                gather_desc(c, k).wait()
                scale_half(bufs[k], 0)
                out_half_desc(c, k, 0).start()
                scale_half(bufs[k], 1)
                out_half_desc(c, k, 1).start()
                k2 = (k + 2) % NBUF
                if k < NBUF - 2:
                    @pl.when(g >= 1)
                    def _():
                        out_desc(c + 2 - NBUF, k2).wait()
                    gather_desc(c + 2, k2).start()
                else:
                    @pl.when(g < NGRP - 1)
                    def _():
                        out_desc(c + 2 - NBUF, k2).wait()
                        gather_desc(c + 2, k2).start()
            return ()

        lax.fori_loop(0, NGRP, group, ())
        for j in range(NBUF):
            c = NCHUNK - NBUF + j
            out_desc(c, c % NBUF).wait()

    return body(table, x)


def kernel(x, table):
    R, C = x.shape
    D = table.shape[1]
    if x.dtype != jnp.int32:
        x = x.astype(jnp.int32)
    out = _gather_scale(table, x, R, C, D)
    return out.reshape(R, C, D)


# speedup vs baseline: 1.0495x; 1.0495x over previous
import functools
import math

import jax
import jax.numpy as jnp
from jax import lax
from jax.experimental import pallas as pl
from jax.experimental.pallas import tpu as pltpu
from jax.experimental.pallas import tpu_sc as plsc

NC = 2
NS = 16
NW = NC * NS
L = 16

D_MODEL = 1024
SCALE = math.sqrt(D_MODEL)

CHUNK = 16
NBUF = 4


@functools.partial(jax.jit, static_argnums=(2, 3, 4))
def _gather_scale(table, x, R, C, D):
    B = R * C
    b_per_w = B // NW
    w_per_r = C // b_per_w
    NCHUNK = b_per_w // CHUNK
    NGRP = NCHUNK // NBUF
    mesh = plsc.VectorSubcoreMesh(core_axis_name="c", subcore_axis_name="s")

    @functools.partial(
        pl.kernel,
        out_type=jax.ShapeDtypeStruct((B, D), jnp.float32),
        mesh=mesh,
        scratch_types=(
            [pltpu.VMEM((b_per_w,), jnp.int32)]
            + [pltpu.VMEM((CHUNK, D), jnp.float32)] * NBUF
            + [pltpu.SemaphoreType.DMA] * (2 * NBUF + 2)
        ),
    )
    def body(table_hbm, idx_hbm, out_hbm, idx_v, *rest):
        bufs = rest[:NBUF]
        gsems = rest[NBUF:2 * NBUF]
        osems = rest[2 * NBUF:3 * NBUF]
        isem0, isem1 = rest[3 * NBUF:]
        wid = lax.axis_index("s") * NC + lax.axis_index("c")
        base = wid * b_per_w
        xr = wid // w_per_r
        xc = (wid % w_per_r) * b_per_w
        HALF = b_per_w // 2
        di0 = pltpu.make_async_copy(
            idx_hbm.at[xr, pl.ds(xc, HALF)], idx_v.at[pl.ds(0, HALF)], isem0)
        di1 = pltpu.make_async_copy(
            idx_hbm.at[xr, pl.ds(xc + HALF, HALF)],
            idx_v.at[pl.ds(HALF, HALF)], isem1)
        di0.start()
        di1.start()

        def gather_desc(c, k):
            return pltpu.make_async_copy(
                table_hbm.at[idx_v.at[pl.ds(c * CHUNK, CHUNK)]],
                bufs[k], gsems[k])

        def out_desc(c, k):
            return pltpu.make_async_copy(
                bufs[k], out_hbm.at[pl.ds(base + c * CHUNK, CHUNK)], osems[k])

        def scale_buf(buf):
            @plsc.parallel_loop(0, CHUNK)
            def _(r):
                for j in range(D // L):
                    sl = (r, pl.ds(j * L, L))
                    buf[sl] = buf[sl] * SCALE

        di0.wait()
        gather_desc(0, 0).start()
        gather_desc(1, 1).start()
        di1.wait()

        def group(g, _):
            for k in range(NBUF):
                c = g * NBUF + k
                gather_desc(c, k).wait()
                scale_buf(bufs[k])
                out_desc(c, k).start()
                k2 = (k + 2) % NBUF
                if k < NBUF - 2:
                    @pl.when(g >= 1)
                    def _():
                        out_desc(c + 2 - NBUF, k2).wait()
                    gather_desc(c + 2, k2).start()
                else:
                    @pl.when(g < NGRP - 1)
                    def _():
                        out_desc(c + 2 - NBUF, k2).wait()
                        gather_desc(c + 2, k2).start()
            return ()

        lax.fori_loop(0, NGRP, group, ())
        for j in range(NBUF):
            c = NCHUNK - NBUF + j
            out_desc(c, c % NBUF).wait()

    return body(table, x)


def kernel(x, table):
    R, C = x.shape
    D = table.shape[1]
    if x.dtype != jnp.int32:
        x = x.astype(jnp.int32)
    out = _gather_scale(table, x, R, C, D)
    return out.reshape(R, C, D)


# speedup vs baseline: 1.0697x; 1.0193x over previous
import functools
import math

import jax
import jax.numpy as jnp
from jax import lax
from jax.experimental import pallas as pl
from jax.experimental.pallas import tpu as pltpu
from jax.experimental.pallas import tpu_sc as plsc

NC = 2
NS = 16
NW = NC * NS
L = 16

D_MODEL = 1024
SCALE = math.sqrt(D_MODEL)

CHUNK = 16
NBUF = 4


@functools.partial(jax.jit, static_argnums=(2, 3, 4))
def _gather_scale(table, x, R, C, D):
    B = R * C
    b_per_w = B // NW
    w_per_r = C // b_per_w
    NCHUNK = b_per_w // CHUNK
    NGRP = NCHUNK // NBUF
    mesh = plsc.VectorSubcoreMesh(core_axis_name="c", subcore_axis_name="s")

    @functools.partial(
        pl.kernel,
        out_type=jax.ShapeDtypeStruct((B, D), jnp.float32),
        mesh=mesh,
        scratch_types=(
            [pltpu.VMEM((b_per_w,), jnp.int32)]
            + [pltpu.VMEM((CHUNK, D), jnp.float32)] * NBUF
            + [pltpu.SemaphoreType.DMA] * (2 * NBUF + 2)
        ),
    )
    def body(table_hbm, idx_hbm, out_hbm, idx_v, *rest):
        bufs = rest[:NBUF]
        gsems = rest[NBUF:2 * NBUF]
        osems = rest[2 * NBUF:3 * NBUF]
        isem0, isem1 = rest[3 * NBUF:]
        wid = lax.axis_index("s") * NC + lax.axis_index("c")
        base = wid * b_per_w
        xr = wid // w_per_r
        xc = (wid % w_per_r) * b_per_w
        HALF = b_per_w // 2
        di0 = pltpu.make_async_copy(
            idx_hbm.at[xr, pl.ds(xc, HALF)], idx_v.at[pl.ds(0, HALF)], isem0)
        di1 = pltpu.make_async_copy(
            idx_hbm.at[xr, pl.ds(xc + HALF, HALF)],
            idx_v.at[pl.ds(HALF, HALF)], isem1)
        di0.start()
        di1.start()

        def gather_desc(c, k):
            return pltpu.make_async_copy(
                table_hbm.at[idx_v.at[pl.ds(c * CHUNK, CHUNK)]],
                bufs[k], gsems[k])

        def out_desc(c, k):
            return pltpu.make_async_copy(
                bufs[k], out_hbm.at[pl.ds(base + c * CHUNK, CHUNK)], osems[k])

        def scale_buf(buf):
            @plsc.parallel_loop(0, CHUNK)
            def _(r):
                for j in range(D // L):
                    sl = (r, pl.ds(j * L, L))
                    buf[sl] = buf[sl] * SCALE

        di0.wait()
        gather_desc(0, 0).start()
        gather_desc(1, 1).start()
        di1.wait()

        def group(g, _):
            for k in range(NBUF):
                c = g * NBUF + k
                gather_desc(c, k).wait()
                k2 = (k + 2) % NBUF
                if k < NBUF - 2:
                    @pl.when(g >= 1)
                    def _():
                        out_desc(c + 2 - NBUF, k2).wait()
                    gather_desc(c + 2, k2).start()
                else:
                    @pl.when(g < NGRP - 1)
                    def _():
                        out_desc(c + 2 - NBUF, k2).wait()
                        gather_desc(c + 2, k2).start()
                scale_buf(bufs[k])
                out_desc(c, k).start()
            return ()

        lax.fori_loop(0, NGRP, group, ())
        for j in range(NBUF):
            c = NCHUNK - NBUF + j
            out_desc(c, c % NBUF).wait()

    return body(table, x)


def kernel(x, table):
    R, C = x.shape
    D = table.shape[1]
    if x.dtype != jnp.int32:
        x = x.astype(jnp.int32)
    out = _gather_scale(table, x, R, C, D)
    return out.reshape(R, C, D)
